# Initial kernel scaffold; baseline (speedup 1.0000x reference)
import jax
import jax.numpy as jnp
from jax.experimental import pallas as pl


def kernel(x, adj, tar_ei, NCN_mode, W_xslin, b_xslin):
    raise NotImplementedError("write your pallas kernel here")


# speedup vs baseline: 3.9551x; 3.9551x over previous
import functools

import jax
import jax.numpy as jnp
from jax import lax
from jax.experimental import pallas as pl
from jax.experimental.pallas import tpu as pltpu
from jax.experimental.pallas import tpu_sc as plsc

NC = 2
NS = 16
NW = NC * NS
D = 256
CAP = 2048
LCH = 128
RS_PAD = 10240


def _sread(ref, idx):
    return ref[pl.ds(idx, 16)][0]


def _sc_body(n_targets, x_hbm, dst_hbm, rs_hbm, ti_hbm, tj_hbm, w1_hbm, w2_hbm,
             out_hbm, rs_v, ib_v, jb_v, xi_v, xj_v, xv_v, w1_v, w2_v,
             ti_v, tj_v, out_v, red_v):
    ntpw = n_targets // NW
    wid = lax.axis_index("s") * NC + lax.axis_index("c")
    base = wid * ntpw
    pltpu.sync_copy(rs_hbm, rs_v)
    pltpu.sync_copy(ti_hbm.at[pl.ds(base, ntpw)], ti_v.at[pl.ds(0, ntpw)])
    pltpu.sync_copy(tj_hbm.at[pl.ds(base, ntpw)], tj_v.at[pl.ds(0, ntpw)])
    pltpu.sync_copy(w1_hbm, w1_v)
    pltpu.sync_copy(w2_hbm, w2_v)
    iota = lax.iota(jnp.int32, 16)

    def per_target(t, ovec):
        i = _sread(ti_v, t)
        j = _sread(tj_v, t)
        rsi = _sread(rs_v, i)
        rsj = _sread(rs_v, j)
        ki = jnp.minimum(_sread(rs_v, i + 1) - rsi, CAP)
        kj = jnp.minimum(_sread(rs_v, j + 1) - rsj, CAP)
        pltpu.sync_copy(x_hbm.at[i], xi_v)
        pltpu.sync_copy(x_hbm.at[j], xj_v)
        fbi = (rsi // 8) * 8
        shi = rsi - fbi
        fbj = (rsj // 8) * 8
        shj = rsj - fbj

        def cp_i(c, _):
            pltpu.sync_copy(dst_hbm.at[pl.ds(fbi + c * LCH, LCH)],
                            ib_v.at[pl.ds(c * LCH, LCH)])
            return 0

        def cp_j(c, _):
            pltpu.sync_copy(dst_hbm.at[pl.ds(fbj + c * LCH, LCH)],
                            jb_v.at[pl.ds(c * LCH, LCH)])
            return 0

        lax.fori_loop(0, (ki + shi + LCH - 1) // LCH, cp_i, 0)
        lax.fori_loop(0, (kj + shj + LCH - 1) // LCH, cp_j, 0)

        acc = jnp.zeros((16,), jnp.float32)
        for cc in range(D // 16):
            sl = pl.ds(cc * 16, 16)
            acc = acc + xi_v[sl] * xj_v[sl] * w1_v[sl]

        def per_chunk(c, acc):
            gpos = c * 16 + iota
            idx = shj + gpos
            jv = plsc.load_gather(jb_v, [idx])
            prev = plsc.load_gather(jb_v, [jnp.maximum(idx - 1, 0)])
            focc = (jv != prev) | (gpos == 0)
            lo = jnp.zeros((16,), jnp.int32)
            hi = jnp.full((16,), ki, jnp.int32)
            for _ in range(11):
                mid = (lo + hi) // 2
                mv = plsc.load_gather(ib_v, [shi + mid])
                ltv = mv < jv
                lo = jnp.where(ltv, mid + 1, lo)
                hi = jnp.where(ltv, hi, mid)
            fv = plsc.load_gather(ib_v, [shi + jnp.minimum(lo, CAP - 1)])
            member = (gpos < kj) & (lo < ki) & (fv == jv) & focc

            def m_cond(st):
                m, _ = st
                p = plsc.all_reduce_population_count(m)
                p = p if p.ndim == 0 else p[0]
                return p > 0

            def m_body(st):
                m, a = st
                r = plsc.all_reduce_ffs(m)
                l = r if r.ndim == 0 else r[0]
                v = _sread(jb_v, shj + c * 16 + l)
                pltpu.sync_copy(x_hbm.at[v], xv_v)
                for cc in range(D // 16):
                    sl = pl.ds(cc * 16, 16)
                    a = a + xv_v[sl] * w2_v[sl]
                return m & (iota != l), a

            member, acc = lax.while_loop(m_cond, m_body, (member, acc))
            return acc

        acc = lax.fori_loop(0, (kj + 15) // 16, per_chunk, acc)
        for sh in (8, 4, 2, 1):
            red_v[...] = acc
            acc = acc + plsc.load_gather(red_v, [iota ^ sh])
        return jnp.where(iota == t % 16, acc, ovec)

    def per_group(g, _):
        ovec = lax.fori_loop(g * 16, g * 16 + 16, per_target,
                             jnp.zeros((16,), jnp.float32))
        out_v[pl.ds(g * 16, 16)] = ovec
        return 0

    lax.fori_loop(0, ntpw // 16, per_group, 0)
    pltpu.sync_copy(out_v, out_hbm.at[pl.ds(base, ntpw)])


@functools.partial(jax.jit, static_argnames=("n_nodes", "n_targets"))
def _ncn_sc(x, dst_pad, rs_pad, ti, tj, w1, w2, n_nodes, n_targets):
    ntpw = n_targets // NW
    mesh = plsc.VectorSubcoreMesh(core_axis_name="c", subcore_axis_name="s")
    f = pl.kernel(
        functools.partial(_sc_body, n_targets),
        out_type=jax.ShapeDtypeStruct((n_targets,), jnp.float32),
        mesh=mesh,
        scratch_types=[
            pltpu.VMEM((RS_PAD,), jnp.int32),
            pltpu.VMEM((CAP + LCH,), jnp.int32),
            pltpu.VMEM((CAP + LCH,), jnp.int32),
            pltpu.VMEM((D,), jnp.float32),
            pltpu.VMEM((D,), jnp.float32),
            pltpu.VMEM((D,), jnp.float32),
            pltpu.VMEM((D,), jnp.float32),
            pltpu.VMEM((D,), jnp.float32),
            pltpu.VMEM((ntpw + 16,), jnp.int32),
            pltpu.VMEM((ntpw + 16,), jnp.int32),
            pltpu.VMEM((ntpw,), jnp.float32),
            pltpu.VMEM((16,), jnp.float32),
        ],
        compiler_params=pltpu.CompilerParams(needs_layout_passes=False),
    )
    return f(x, dst_pad, rs_pad, ti, tj, w1, w2)


def kernel(x, adj, tar_ei, NCN_mode, W_xslin, b_xslin):
    n_nodes, d = x.shape
    n_edges = adj.shape[1]
    n_targets = tar_ei.shape[1]
    w1 = W_xslin[0, :d]
    w2 = W_xslin[0, d:]
    keys = jnp.sort(adj[0] * n_nodes + adj[1])
    dst_pad = jnp.concatenate(
        [keys % n_nodes, jnp.full((LCH + 64,), n_nodes, jnp.int32)])
    counts = jnp.zeros((n_nodes,), jnp.int32).at[adj[0]].add(1)
    row_start = jnp.concatenate(
        [jnp.zeros((1,), jnp.int32), jnp.cumsum(counts, dtype=jnp.int32)])
    rs_pad = jnp.concatenate(
        [row_start, jnp.full((RS_PAD - n_nodes - 1,), n_edges, jnp.int32)])
    raw = _ncn_sc(x, dst_pad, rs_pad, tar_ei[0], tar_ei[1], w1, w2,
                  n_nodes=n_nodes, n_targets=n_targets)
    mode_ok = jnp.asarray(jnp.asarray(NCN_mode) == 1, x.dtype)
    return ((raw + b_xslin[0]) * mode_ok).reshape(n_targets, 1)


# speedup vs baseline: 5.6932x; 1.4395x over previous
import functools

import jax
import jax.numpy as jnp
from jax import lax
from jax.experimental import pallas as pl
from jax.experimental.pallas import tpu as pltpu
from jax.experimental.pallas import tpu_sc as plsc

NC = 2
NS = 16
NW = NC * NS
D = 256
CAP = 2048
LCH = 128
RS_PAD = 10240


def _sread(ref, idx):
    return ref[pl.ds(idx, 16)][0]


def _sc_body(n_targets, x_hbm, dst_hbm, rs_hbm, ti_hbm, tj_hbm, w1_hbm, w2_hbm,
             out_hbm, rs_v, ib_v, jb_v, xia_v, xja_v, lia_v, lja_v, xv_v,
             w1_v, w2_v, ti_v, tj_v, out_v, red_v, sem):
    ntpw = n_targets // NW
    wid = lax.axis_index("s") * NC + lax.axis_index("c")
    base = wid * ntpw
    pltpu.sync_copy(rs_hbm, rs_v)
    pltpu.sync_copy(ti_hbm.at[pl.ds(base, ntpw)], ti_v.at[pl.ds(0, ntpw)])
    pltpu.sync_copy(tj_hbm.at[pl.ds(base, ntpw)], tj_v.at[pl.ds(0, ntpw)])
    pltpu.sync_copy(w1_hbm, w1_v)
    pltpu.sync_copy(w2_hbm, w2_v)
    iota = lax.iota(jnp.int32, 16)

    def prefetch(t, _):
        i = _sread(ti_v, t)
        j = _sread(tj_v, t)
        fbi = (_sread(rs_v, i) // 8) * 8
        fbj = (_sread(rs_v, j) // 8) * 8
        pltpu.async_copy(x_hbm.at[i], xia_v.at[pl.ds(t * D, D)], sem)
        pltpu.async_copy(x_hbm.at[j], xja_v.at[pl.ds(t * D, D)], sem)
        pltpu.async_copy(dst_hbm.at[pl.ds(fbi, LCH)],
                         lia_v.at[pl.ds(t * LCH, LCH)], sem)
        pltpu.async_copy(dst_hbm.at[pl.ds(fbj, LCH)],
                         lja_v.at[pl.ds(t * LCH, LCH)], sem)
        return 0

    lax.fori_loop(0, ntpw, prefetch, 0)

    def drain(t, _):
        pltpu.make_async_copy(x_hbm.at[0], xia_v.at[pl.ds(t * D, D)], sem).wait()
        pltpu.make_async_copy(x_hbm.at[0], xja_v.at[pl.ds(t * D, D)], sem).wait()
        pltpu.make_async_copy(dst_hbm.at[pl.ds(0, LCH)],
                              lia_v.at[pl.ds(t * LCH, LCH)], sem).wait()
        pltpu.make_async_copy(dst_hbm.at[pl.ds(0, LCH)],
                              lja_v.at[pl.ds(t * LCH, LCH)], sem).wait()
        return 0

    lax.fori_loop(0, ntpw, drain, 0)

    def _intersect(acc, jref, jbase, shj, kj, iref, ibase, shi, ki, niter):
        def per_chunk(c, acc):
            gpos = c * 16 + iota
            idx = jbase + shj + gpos
            jv = plsc.load_gather(jref, [idx])
            prev = plsc.load_gather(jref, [jnp.maximum(idx - 1, 0)])
            focc = (jv != prev) | (gpos == 0)
            lo = jnp.zeros((16,), jnp.int32)
            hi = jnp.full((16,), ki, jnp.int32)
            for _ in range(niter):
                mid = (lo + hi) // 2
                mv = plsc.load_gather(iref, [ibase + shi + mid])
                ltv = mv < jv
                lo = jnp.where(ltv, mid + 1, lo)
                hi = jnp.where(ltv, hi, mid)
            fv = plsc.load_gather(iref, [ibase + shi + lo])
            member = (gpos < kj) & (lo < ki) & (fv == jv) & focc

            def m_cond(st):
                m, _ = st
                p = plsc.all_reduce_population_count(m)
                p = p if p.ndim == 0 else p[0]
                return p > 0

            def m_body(st):
                m, a = st
                r = plsc.all_reduce_ffs(m)
                l = r if r.ndim == 0 else r[0]
                v = _sread(jref, jbase + shj + c * 16 + l)
                pltpu.sync_copy(x_hbm.at[v], xv_v)
                for cc in range(D // 16):
                    sl = pl.ds(cc * 16, 16)
                    a = a + xv_v[sl] * w2_v[sl]
                return m & (iota != l), a

            member, acc = lax.while_loop(m_cond, m_body, (member, acc))
            return acc

        return lax.fori_loop(0, (kj + 15) // 16, per_chunk, acc)

    def per_target(t, ovec):
        i = _sread(ti_v, t)
        j = _sread(tj_v, t)
        rsi = _sread(rs_v, i)
        rsj = _sread(rs_v, j)
        ki = _sread(rs_v, i + 1) - rsi
        kj = _sread(rs_v, j + 1) - rsj
        shi = rsi - (rsi // 8) * 8
        shj = rsj - (rsj // 8) * 8

        acc = jnp.zeros((16,), jnp.float32)
        for cc in range(D // 16):
            a_sl = pl.ds(t * D + cc * 16, 16)
            w_sl = pl.ds(cc * 16, 16)
            acc = acc + xia_v[a_sl] * xja_v[a_sl] * w1_v[w_sl]

        def fast(acc):
            return _intersect(acc, lja_v, t * LCH, shj, kj,
                              lia_v, t * LCH, shi, ki, 8)

        def slow(acc):
            kic = jnp.minimum(ki, CAP)
            kjc = jnp.minimum(kj, CAP)

            def cp_i(c, _):
                pltpu.sync_copy(dst_hbm.at[pl.ds(rsi - shi + c * LCH, LCH)],
                                ib_v.at[pl.ds(c * LCH, LCH)])
                return 0

            def cp_j(c, _):
                pltpu.sync_copy(dst_hbm.at[pl.ds(rsj - shj + c * LCH, LCH)],
                                jb_v.at[pl.ds(c * LCH, LCH)])
                return 0

            lax.fori_loop(0, (kic + shi + LCH - 1) // LCH, cp_i, 0)
            lax.fori_loop(0, (kjc + shj + LCH - 1) // LCH, cp_j, 0)
            return _intersect(acc, jb_v, 0, shj, kjc, ib_v, 0, shi, kic, 12)

        overflow = (shi + ki > LCH) | (shj + kj > LCH)
        acc = lax.cond(overflow, slow, fast, acc)

        for sh in (8, 4, 2, 1):
            red_v[...] = acc
            acc = acc + plsc.load_gather(red_v, [iota ^ sh])
        return jnp.where(iota == t % 16, acc, ovec)

    def per_group(g, _):
        ovec = lax.fori_loop(g * 16, g * 16 + 16, per_target,
                             jnp.zeros((16,), jnp.float32))
        out_v[pl.ds(g * 16, 16)] = ovec
        return 0

    lax.fori_loop(0, ntpw // 16, per_group, 0)
    pltpu.sync_copy(out_v, out_hbm.at[pl.ds(base, ntpw)])


@functools.partial(jax.jit, static_argnames=("n_nodes", "n_targets"))
def _ncn_sc(x, dst_pad, rs_pad, ti, tj, w1, w2, n_nodes, n_targets):
    ntpw = n_targets // NW
    mesh = plsc.VectorSubcoreMesh(core_axis_name="c", subcore_axis_name="s")
    f = pl.kernel(
        functools.partial(_sc_body, n_targets),
        out_type=jax.ShapeDtypeStruct((n_targets,), jnp.float32),
        mesh=mesh,
        scratch_types=[
            pltpu.VMEM((RS_PAD,), jnp.int32),
            pltpu.VMEM((CAP + LCH,), jnp.int32),
            pltpu.VMEM((CAP + LCH,), jnp.int32),
            pltpu.VMEM((ntpw * D + 16,), jnp.float32),
            pltpu.VMEM((ntpw * D + 16,), jnp.float32),
            pltpu.VMEM((ntpw * LCH + 16,), jnp.int32),
            pltpu.VMEM((ntpw * LCH + 16,), jnp.int32),
            pltpu.VMEM((D,), jnp.float32),
            pltpu.VMEM((D,), jnp.float32),
            pltpu.VMEM((D,), jnp.float32),
            pltpu.VMEM((ntpw + 16,), jnp.int32),
            pltpu.VMEM((ntpw + 16,), jnp.int32),
            pltpu.VMEM((ntpw,), jnp.float32),
            pltpu.VMEM((16,), jnp.float32),
            pltpu.SemaphoreType.DMA,
        ],
        compiler_params=pltpu.CompilerParams(needs_layout_passes=False),
    )
    return f(x, dst_pad, rs_pad, ti, tj, w1, w2)


def kernel(x, adj, tar_ei, NCN_mode, W_xslin, b_xslin):
    n_nodes, d = x.shape
    n_edges = adj.shape[1]
    n_targets = tar_ei.shape[1]
    w1 = W_xslin[0, :d]
    w2 = W_xslin[0, d:]
    keys = jnp.sort(adj[0] * n_nodes + adj[1])
    dst_pad = jnp.concatenate(
        [keys % n_nodes, jnp.full((LCH + 64,), n_nodes, jnp.int32)])
    counts = jnp.zeros((n_nodes,), jnp.int32).at[adj[0]].add(1)
    row_start = jnp.concatenate(
        [jnp.zeros((1,), jnp.int32), jnp.cumsum(counts, dtype=jnp.int32)])
    rs_pad = jnp.concatenate(
        [row_start, jnp.full((RS_PAD - n_nodes - 1,), n_edges, jnp.int32)])
    raw = _ncn_sc(x, dst_pad, rs_pad, tar_ei[0], tar_ei[1], w1, w2,
                  n_nodes=n_nodes, n_targets=n_targets)
    mode_ok = jnp.asarray(jnp.asarray(NCN_mode) == 1, x.dtype)
    return ((raw + b_xslin[0]) * mode_ok).reshape(n_targets, 1)
